# Initial kernel scaffold; baseline (speedup 1.0000x reference)
import jax
import jax.numpy as jnp
from jax.experimental import pallas as pl


def kernel(t, atom_types, frac_coords, lattices, num_atoms, node2graph, node_emb_w, node_emb_b, latent_w, latent_b, edge_w1, edge_b1, edge_w2, edge_b2, node_w1, node_b1, node_w2, node_b2, coord_w, lattice_w):
    raise NotImplementedError("write your pallas kernel here")


# speedup vs baseline: 23.2953x; 23.2953x over previous
import functools

import jax
import jax.numpy as jnp
import numpy as np
from jax.experimental import pallas as pl
from jax.experimental.pallas import tpu as pltpu


def _silu(x):
    h = 0.5 * x
    return h + h * jnp.tanh(h)


def _fused_kernel(nat, n_freq, n_layers,
                  t_ref, at_ref, frac_ref, lat_ref, cnt_ref,
                  new_ref, neb_ref, lw_ref, lb_ref,
                  ew1_ref, eb1_ref, ew2_ref, eb2_ref,
                  nw1_ref, nb1_ref, nw2_ref, nb2_ref,
                  cw_ref, latw_ref,
                  latout_ref, coord_ref):
    cg = t_ref.shape[0]
    hidden = new_ref.shape[1]
    latent = t_ref.shape[2]
    nn = cg * nat
    ne = nn * nat

    f32 = jnp.float32
    dot = functools.partial(jnp.dot, preferred_element_type=f32)

    at = at_ref[...].reshape(nn, at_ref.shape[2])
    h0 = dot(at, new_ref[...]) + neb_ref[...]
    lw = lw_ref[...]
    t_blk = t_ref[...].reshape(cg, latent)
    t_term = dot(t_blk, lw[hidden:, :])
    h = dot(h0, lw[:hidden, :]) \
        + jnp.broadcast_to(t_term[:, None, :], (cg, nat, hidden)).reshape(nn, hidden) \
        + lb_ref[...]

    lat = lat_ref[...]
    lat_ip = jnp.sum(lat[:, :, None, :] * lat[:, None, :, :], axis=-1)
    lat_ip = lat_ip.reshape(cg, 9)

    frac = frac_ref[...].reshape(nn, 3)
    freqs = 2.0 * np.pi * jax.lax.broadcasted_iota(jnp.int32, (1, 1, n_freq), 2).astype(f32)
    femb = (frac[:, :, None] * freqs).reshape(nn, 3 * n_freq)
    s_n = jnp.sin(femb)
    c_n = jnp.cos(femb)
    x_n = jnp.concatenate([c_n, s_n, c_n, s_n], axis=-1).reshape(cg, nat, 1, 12 * n_freq)
    y_n = jnp.concatenate([s_n, c_n, c_n, s_n], axis=-1).reshape(cg, 1, nat, 12 * n_freq)
    pmat = (x_n * y_n).reshape(ne, 12 * n_freq)

    inv_cnt = 1.0 / cnt_ref[...].reshape(cg)

    for i in range(n_layers):
        w1 = ew1_ref[i]
        a_term = dot(h, w1[:hidden, :]) + eb1_ref[i]
        b_term = dot(h, w1[hidden:2 * hidden, :])
        g_term = dot(lat_ip, w1[2 * hidden:2 * hidden + 9, :])
        wsin = w1[2 * hidden + 9:2 * hidden + 9 + 3 * n_freq, :]
        wcos = w1[2 * hidden + 9 + 3 * n_freq:, :]
        wdis = jnp.concatenate([wsin, -wsin, wcos, wcos], axis=0)
        d_term = dot(pmat, wdis)
        bg_term = b_term.reshape(cg, nat, hidden) + g_term.reshape(cg, 1, hidden)
        pre = (a_term.reshape(cg, nat, 1, hidden)
               + bg_term.reshape(cg, 1, nat, hidden)
               + d_term.reshape(cg, nat, nat, hidden))
        ef = _silu(pre).reshape(ne, hidden)
        ef = _silu(dot(ef, ew2_ref[i]) + eb2_ref[i])
        agg = ef.reshape(cg, nat, nat, hidden).sum(axis=2)
        agg = (agg * inv_cnt[:, None, None]).reshape(nn, hidden)
        nw1 = nw1_ref[i]
        out = _silu(dot(h, nw1[:hidden, :]) + dot(agg, nw1[hidden:, :]) + nb1_ref[i])
        out = _silu(dot(out, nw2_ref[i]) + nb2_ref[i])
        h = h + out

    coord_ref[...] = dot(h, cw_ref[...]).reshape(cg, nat, 3)
    gf = h.reshape(cg, nat, hidden).sum(axis=1) * (1.0 / float(nat))
    lo = dot(gf, latw_ref[...]).reshape(cg, 3, 3)
    latout_ref[...] = jnp.sum(lo[:, :, :, None] * lat[:, None, :, :], axis=2)


def kernel(t, atom_types, frac_coords, lattices, num_atoms, node2graph,
           node_emb_w, node_emb_b, latent_w, latent_b,
           edge_w1, edge_b1, edge_w2, edge_b2,
           node_w1, node_b1, node_w2, node_b2,
           coord_w, lattice_w):
    n_nodes = frac_coords.shape[0]
    n_graphs = t.shape[0]
    nat = n_nodes // n_graphs
    hidden = node_emb_w.shape[1]
    latent = t.shape[1]
    max_atoms = atom_types.shape[1]
    n_layers = edge_w1.shape[0]
    dis_dim = edge_w1.shape[1] - 2 * hidden - 9
    n_freq = dis_dim // 6

    cg = 25
    while n_graphs % cg != 0:
        cg -= 1
    grid = (n_graphs // cg,)

    t3 = t.reshape(n_graphs, 1, latent)
    at3 = atom_types.reshape(n_graphs, nat, max_atoms)
    frac3 = frac_coords.reshape(n_graphs, nat, 3)
    cnt3 = jnp.maximum(num_atoms.astype(jnp.float32), 1.0).reshape(n_graphs, 1, 1)
    neb2 = node_emb_b.reshape(1, hidden)
    lb2 = latent_b.reshape(1, hidden)

    def blk(shape):
        return pl.BlockSpec(shape, lambda i: (i,) + (0,) * (len(shape) - 1))

    def whole(shape):
        return pl.BlockSpec(shape, lambda i: (0,) * len(shape))

    lattice_out, coord3 = pl.pallas_call(
        functools.partial(_fused_kernel, nat, n_freq, n_layers),
        grid=grid,
        compiler_params=pltpu.CompilerParams(
            dimension_semantics=("parallel",)),
        in_specs=[
            blk((cg, 1, latent)),
            blk((cg, nat, max_atoms)),
            blk((cg, nat, 3)),
            blk((cg, 3, 3)),
            blk((cg, 1, 1)),
            whole(node_emb_w.shape),
            whole(neb2.shape),
            whole(latent_w.shape),
            whole(lb2.shape),
            whole(edge_w1.shape),
            whole(edge_b1.shape),
            whole(edge_w2.shape),
            whole(edge_b2.shape),
            whole(node_w1.shape),
            whole(node_b1.shape),
            whole(node_w2.shape),
            whole(node_b2.shape),
            whole(coord_w.shape),
            whole(lattice_w.shape),
        ],
        out_specs=[
            blk((cg, 3, 3)),
            blk((cg, nat, 3)),
        ],
        out_shape=[
            jax.ShapeDtypeStruct((n_graphs, 3, 3), jnp.float32),
            jax.ShapeDtypeStruct((n_graphs, nat, 3), jnp.float32),
        ],
        interpret=False,
    )(t3, at3, frac3, lattices, cnt3,
      node_emb_w, neb2, latent_w, lb2,
      edge_w1, edge_b1, edge_w2, edge_b2,
      node_w1, node_b1, node_w2, node_b2,
      coord_w, lattice_w)

    return (lattice_out, coord3.reshape(n_nodes, 3))


# speedup vs baseline: 23.3369x; 1.0018x over previous
import functools

import jax
import jax.numpy as jnp
import numpy as np
from jax.experimental import pallas as pl
from jax.experimental.pallas import tpu as pltpu


def _silu(x):
    h = 0.5 * x
    return h + h * jnp.tanh(h)


def _silu_of_double(h):
    return h + h * jnp.tanh(h)


def _fused_kernel(nat, n_freq, n_layers,
                  t_ref, at_ref, frac_ref, lat_ref, cnt_ref,
                  new_ref, neb_ref, lw_ref, lb_ref,
                  ew1_ref, eb1_ref, ew2_ref, eb2_ref,
                  nw1_ref, nb1_ref, nw2_ref, nb2_ref,
                  cw_ref, latw_ref,
                  latout_ref, coord_ref):
    cg = t_ref.shape[0]
    hidden = new_ref.shape[1]
    latent = t_ref.shape[2]
    nn = cg * nat
    ne = nn * nat

    f32 = jnp.float32
    dot = functools.partial(jnp.dot, preferred_element_type=f32)

    at = at_ref[...].reshape(nn, at_ref.shape[2])
    h0 = dot(at, new_ref[...]) + neb_ref[...]
    lw = lw_ref[...]
    t_blk = t_ref[...].reshape(cg, latent)
    t_term = dot(t_blk, lw[hidden:, :])
    rep = (jax.lax.broadcasted_iota(jnp.int32, (nn, cg), 0) // nat
           == jax.lax.broadcasted_iota(jnp.int32, (nn, cg), 1)).astype(f32)
    h = dot(h0, lw[:hidden, :]) + dot(rep, t_term) + lb_ref[...]

    lat = lat_ref[...]
    lat_ip = jnp.sum(lat[:, :, None, :] * lat[:, None, :, :], axis=-1)
    lat_ip = lat_ip.reshape(cg, 9)

    frac = frac_ref[...].reshape(nn, 3)
    freqs = 2.0 * np.pi * jax.lax.broadcasted_iota(jnp.int32, (1, 1, n_freq), 2).astype(f32)
    femb = (frac[:, :, None] * freqs).reshape(nn, 3 * n_freq)
    s_n = jnp.sin(femb)
    c_n = jnp.cos(femb)
    x_n = jnp.concatenate([c_n, s_n, c_n, s_n], axis=-1).reshape(cg, 1, nat, 12 * n_freq)
    y_n = jnp.concatenate([s_n, c_n, c_n, s_n], axis=-1).reshape(cg, nat, 1, 12 * n_freq)
    pmat = (x_n * y_n).reshape(ne, 12 * n_freq)

    inv_cnt = 1.0 / cnt_ref[...].reshape(cg)

    for i in range(n_layers):
        w1 = ew1_ref[i] * 0.5
        a_term = dot(h, w1[:hidden, :]) + eb1_ref[i] * 0.5
        g_term = dot(lat_ip, w1[2 * hidden:2 * hidden + 9, :])
        a_term = a_term + dot(rep, g_term)
        b_term = dot(h, w1[hidden:2 * hidden, :])
        wsin = w1[2 * hidden + 9:2 * hidden + 9 + 3 * n_freq, :]
        wcos = w1[2 * hidden + 9 + 3 * n_freq:, :]
        wdis = jnp.concatenate([wsin, -wsin, wcos, wcos], axis=0)
        d_term = dot(pmat, wdis)
        pre_h = (a_term.reshape(cg, 1, nat, hidden)
                 + b_term.reshape(cg, nat, 1, hidden)
                 + d_term.reshape(cg, nat, nat, hidden))
        ef = _silu_of_double(pre_h).reshape(ne, hidden)
        ef = _silu_of_double(dot(ef, ew2_ref[i] * 0.5) + eb2_ref[i] * 0.5)
        agg = ef.reshape(cg, nat, nat, hidden).sum(axis=1)
        agg = (agg * inv_cnt[:, None, None]).reshape(nn, hidden)
        nw1 = nw1_ref[i]
        out = _silu(dot(h, nw1[:hidden, :]) + dot(agg, nw1[hidden:, :]) + nb1_ref[i])
        out = _silu(dot(out, nw2_ref[i]) + nb2_ref[i])
        h = h + out

    coord_ref[...] = dot(h, cw_ref[...]).reshape(cg, nat, 3)
    gf = h.reshape(cg, nat, hidden).sum(axis=1) * (1.0 / float(nat))
    lo9 = dot(gf, latw_ref[...])
    lat9 = lat.reshape(cg, 9)
    acc = jnp.zeros((cg, 9), f32)
    m9 = jax.lax.broadcasted_iota(jnp.int32, (cg, 9), 1)
    for j in range(3):
        ia = (m9 // 3) * 3 + j
        ib = (m9 % 3) + 3 * j
        acc = acc + (jnp.take_along_axis(lo9, ia, axis=1)
                     * jnp.take_along_axis(lat9, ib, axis=1))
    latout_ref[...] = acc.reshape(cg, 3, 3)


def kernel(t, atom_types, frac_coords, lattices, num_atoms, node2graph,
           node_emb_w, node_emb_b, latent_w, latent_b,
           edge_w1, edge_b1, edge_w2, edge_b2,
           node_w1, node_b1, node_w2, node_b2,
           coord_w, lattice_w):
    n_nodes = frac_coords.shape[0]
    n_graphs = t.shape[0]
    nat = n_nodes // n_graphs
    hidden = node_emb_w.shape[1]
    latent = t.shape[1]
    max_atoms = atom_types.shape[1]
    n_layers = edge_w1.shape[0]
    dis_dim = edge_w1.shape[1] - 2 * hidden - 9
    n_freq = dis_dim // 6

    cg = 20
    while n_graphs % cg != 0:
        cg -= 1
    grid = (n_graphs // cg,)

    t3 = t.reshape(n_graphs, 1, latent)
    at3 = atom_types.reshape(n_graphs, nat, max_atoms)
    frac3 = frac_coords.reshape(n_graphs, nat, 3)
    cnt3 = jnp.maximum(num_atoms.astype(jnp.float32), 1.0).reshape(n_graphs, 1, 1)
    neb2 = node_emb_b.reshape(1, hidden)
    lb2 = latent_b.reshape(1, hidden)

    def blk(shape):
        return pl.BlockSpec(shape, lambda i: (i,) + (0,) * (len(shape) - 1))

    def whole(shape):
        return pl.BlockSpec(shape, lambda i: (0,) * len(shape))

    lattice_out, coord3 = pl.pallas_call(
        functools.partial(_fused_kernel, nat, n_freq, n_layers),
        grid=grid,
        compiler_params=pltpu.CompilerParams(
            dimension_semantics=("parallel",)),
        in_specs=[
            blk((cg, 1, latent)),
            blk((cg, nat, max_atoms)),
            blk((cg, nat, 3)),
            blk((cg, 3, 3)),
            blk((cg, 1, 1)),
            whole(node_emb_w.shape),
            whole(neb2.shape),
            whole(latent_w.shape),
            whole(lb2.shape),
            whole(edge_w1.shape),
            whole(edge_b1.shape),
            whole(edge_w2.shape),
            whole(edge_b2.shape),
            whole(node_w1.shape),
            whole(node_b1.shape),
            whole(node_w2.shape),
            whole(node_b2.shape),
            whole(coord_w.shape),
            whole(lattice_w.shape),
        ],
        out_specs=[
            blk((cg, 3, 3)),
            blk((cg, nat, 3)),
        ],
        out_shape=[
            jax.ShapeDtypeStruct((n_graphs, 3, 3), jnp.float32),
            jax.ShapeDtypeStruct((n_graphs, nat, 3), jnp.float32),
        ],
        interpret=False,
    )(t3, at3, frac3, lattices, cnt3,
      node_emb_w, neb2, latent_w, lb2,
      edge_w1, edge_b1, edge_w2, edge_b2,
      node_w1, node_b1, node_w2, node_b2,
      coord_w, lattice_w)

    return (lattice_out, coord3.reshape(n_nodes, 3))


# speedup vs baseline: 23.8412x; 1.0216x over previous
import functools

import jax
import jax.numpy as jnp
import numpy as np
from jax.experimental import pallas as pl
from jax.experimental.pallas import tpu as pltpu


def _silu(x):
    h = 0.5 * x
    return h + h * jnp.tanh(h)


def _silu_of_double(h):
    return h + h * jnp.tanh(h)


def _fused_kernel(nat, n_freq, n_layers,
                  t_ref, at_ref, frac_ref, lat_ref, cnt_ref,
                  new_ref, neb_ref, lw_ref, lb_ref,
                  ew1_ref, eb1_ref, ew2_ref, eb2_ref,
                  nw1_ref, nb1_ref, nw2_ref, nb2_ref,
                  cw_ref, latw_ref,
                  latout_ref, coord_ref):
    cg = t_ref.shape[0]
    hidden = new_ref.shape[1]
    latent = t_ref.shape[2]
    nn = cg * nat
    ne = nn * nat

    f32 = jnp.float32
    dot = functools.partial(jnp.dot, preferred_element_type=f32)

    at = at_ref[...].reshape(nn, at_ref.shape[2])
    h0 = dot(at, new_ref[...]) + neb_ref[...]
    lw = lw_ref[...]
    t_blk = t_ref[...].reshape(cg, latent)
    t_term = dot(t_blk, lw[hidden:, :])
    rep = (jax.lax.broadcasted_iota(jnp.int32, (nn, cg), 0) // nat
           == jax.lax.broadcasted_iota(jnp.int32, (nn, cg), 1)).astype(f32)
    h = dot(h0, lw[:hidden, :]) + dot(rep, t_term) + lb_ref[...]

    lat = lat_ref[...]
    lat_ip = jnp.sum(lat[:, :, None, :] * lat[:, None, :, :], axis=-1)
    lat_ip = lat_ip.reshape(cg, 9)

    frac = frac_ref[...].reshape(nn, 3)
    freqs = 2.0 * np.pi * jax.lax.broadcasted_iota(jnp.int32, (1, 1, n_freq), 2).astype(f32)
    femb = (frac[:, :, None] * freqs).reshape(nn, 3 * n_freq)
    s_n = jnp.sin(femb)
    c_n = jnp.cos(femb)
    x_n = jnp.concatenate([c_n, s_n, c_n, s_n], axis=-1).reshape(cg, 1, nat, 12 * n_freq)
    y_n = jnp.concatenate([s_n, c_n, c_n, s_n], axis=-1).reshape(cg, nat, 1, 12 * n_freq)
    pmat = (x_n * y_n).reshape(ne, 12 * n_freq)

    inv_cnt = 1.0 / cnt_ref[...].reshape(cg)

    for i in range(n_layers):
        w1 = ew1_ref[i] * 0.5
        a_term = dot(h, w1[:hidden, :]) + eb1_ref[i] * 0.5
        g_term = dot(lat_ip, w1[2 * hidden:2 * hidden + 9, :])
        a_term = a_term + dot(rep, g_term)
        b_term = dot(h, w1[hidden:2 * hidden, :])
        wsin = w1[2 * hidden + 9:2 * hidden + 9 + 3 * n_freq, :]
        wcos = w1[2 * hidden + 9 + 3 * n_freq:, :]
        wdis = jnp.concatenate([wsin, -wsin, wcos, wcos], axis=0)
        d_term = dot(pmat, wdis)
        pre_h = (a_term.reshape(cg, 1, nat, hidden)
                 + b_term.reshape(cg, nat, 1, hidden)
                 + d_term.reshape(cg, nat, nat, hidden))
        ef = _silu_of_double(pre_h).reshape(ne, hidden)
        ef = _silu_of_double(dot(ef, ew2_ref[i] * 0.5) + eb2_ref[i] * 0.5)
        agg = ef.reshape(cg, nat, nat, hidden).sum(axis=1)
        agg = (agg * inv_cnt[:, None, None]).reshape(nn, hidden)
        nw1 = nw1_ref[i]
        out = _silu(dot(h, nw1[:hidden, :]) + dot(agg, nw1[hidden:, :]) + nb1_ref[i])
        out = _silu(dot(out, nw2_ref[i]) + nb2_ref[i])
        h = h + out

    coord_ref[...] = dot(h, cw_ref[...]).reshape(cg, nat, 3)
    gf = h.reshape(cg, nat, hidden).sum(axis=1) * (1.0 / float(nat))
    lo9 = dot(gf, latw_ref[...])
    lat9 = lat.reshape(cg, 9)
    acc = jnp.zeros((cg, 9), f32)
    m9 = jax.lax.broadcasted_iota(jnp.int32, (cg, 9), 1)
    for j in range(3):
        ia = (m9 // 3) * 3 + j
        ib = (m9 % 3) + 3 * j
        acc = acc + (jnp.take_along_axis(lo9, ia, axis=1)
                     * jnp.take_along_axis(lat9, ib, axis=1))
    latout_ref[...] = acc.reshape(cg, 3, 3)


def kernel(t, atom_types, frac_coords, lattices, num_atoms, node2graph,
           node_emb_w, node_emb_b, latent_w, latent_b,
           edge_w1, edge_b1, edge_w2, edge_b2,
           node_w1, node_b1, node_w2, node_b2,
           coord_w, lattice_w):
    n_nodes = frac_coords.shape[0]
    n_graphs = t.shape[0]
    nat = n_nodes // n_graphs
    hidden = node_emb_w.shape[1]
    latent = t.shape[1]
    max_atoms = atom_types.shape[1]
    n_layers = edge_w1.shape[0]
    dis_dim = edge_w1.shape[1] - 2 * hidden - 9
    n_freq = dis_dim // 6

    cg = 25
    while n_graphs % cg != 0:
        cg -= 1
    grid = (n_graphs // cg,)

    t3 = t.reshape(n_graphs, 1, latent)
    at3 = atom_types.reshape(n_graphs, nat, max_atoms)
    frac3 = frac_coords.reshape(n_graphs, nat, 3)
    cnt3 = jnp.maximum(num_atoms.astype(jnp.float32), 1.0).reshape(n_graphs, 1, 1)
    neb2 = node_emb_b.reshape(1, hidden)
    lb2 = latent_b.reshape(1, hidden)

    def blk(shape):
        return pl.BlockSpec(shape, lambda i: (i,) + (0,) * (len(shape) - 1))

    def whole(shape):
        return pl.BlockSpec(shape, lambda i: (0,) * len(shape))

    lattice_out, coord3 = pl.pallas_call(
        functools.partial(_fused_kernel, nat, n_freq, n_layers),
        grid=grid,
        compiler_params=pltpu.CompilerParams(
            dimension_semantics=("parallel",)),
        in_specs=[
            blk((cg, 1, latent)),
            blk((cg, nat, max_atoms)),
            blk((cg, nat, 3)),
            blk((cg, 3, 3)),
            blk((cg, 1, 1)),
            whole(node_emb_w.shape),
            whole(neb2.shape),
            whole(latent_w.shape),
            whole(lb2.shape),
            whole(edge_w1.shape),
            whole(edge_b1.shape),
            whole(edge_w2.shape),
            whole(edge_b2.shape),
            whole(node_w1.shape),
            whole(node_b1.shape),
            whole(node_w2.shape),
            whole(node_b2.shape),
            whole(coord_w.shape),
            whole(lattice_w.shape),
        ],
        out_specs=[
            blk((cg, 3, 3)),
            blk((cg, nat, 3)),
        ],
        out_shape=[
            jax.ShapeDtypeStruct((n_graphs, 3, 3), jnp.float32),
            jax.ShapeDtypeStruct((n_graphs, nat, 3), jnp.float32),
        ],
        interpret=False,
    )(t3, at3, frac3, lattices, cnt3,
      node_emb_w, neb2, latent_w, lb2,
      edge_w1, edge_b1, edge_w2, edge_b2,
      node_w1, node_b1, node_w2, node_b2,
      coord_w, lattice_w)

    return (lattice_out, coord3.reshape(n_nodes, 3))
